# Initial kernel scaffold; baseline (speedup 1.0000x reference)
import jax
import jax.numpy as jnp
from jax.experimental import pallas as pl


def kernel(class_preds, box_preds, centerness_preds, points, strides):
    raise NotImplementedError("write your pallas kernel here")


# speedup vs baseline: 4.2239x; 4.2239x over previous
import functools

import jax
import jax.numpy as jnp
from jax.experimental import pallas as pl
from jax.experimental.pallas import tpu as pltpu

_NUM_CLASSES = 80
_MAX_DET = 100
_IOU_THR = 0.5
_N = 20000
_ROWS_PER_GROUP = 16
_G = _N // _ROWS_PER_GROUP
_GROUP_FLAT = _ROWS_PER_GROUP * _NUM_CLASSES
_LANES = 128

_PACKED = 88
_BIG_I32 = 2 ** 30


def _fcos_kernel(packed_ref, boxes_out, scores_out, classes_out,
                 fused_ref, sel_smem, ord_smem, flat_smem):
    fused = jnp.sqrt(jax.nn.sigmoid(packed_ref[:, 0:_NUM_CLASSES])
                     * jax.nn.sigmoid(packed_ref[:, _NUM_CLASSES:_NUM_CLASSES + 1]))
    fused_ref[...] = fused
    f3 = fused.reshape(_G, _ROWS_PER_GROUP, _NUM_CLASSES)
    gmax = jnp.max(jnp.max(f3, axis=2), axis=1)
    M0 = gmax.reshape(1, _G)

    lane_g = jax.lax.broadcasted_iota(jnp.int32, (1, _G), 1)
    lane = jax.lax.broadcasted_iota(jnp.int32, (1, _LANES), 1)
    blk_flat_iota = (
        jax.lax.broadcasted_iota(jnp.int32, (_ROWS_PER_GROUP, _NUM_CLASSES), 0)
        * _NUM_CLASSES
        + jax.lax.broadcasted_iota(jnp.int32, (_ROWS_PER_GROUP, _NUM_CLASSES), 1)
    )

    def topk_body(k, carry):
        M, scores_acc, flat_acc = carry
        m = jnp.max(M)
        g = jnp.min(jnp.where(M == m, lane_g, _BIG_I32))
        blk = fused_ref[pl.ds(g * _ROWS_PER_GROUP, _ROWS_PER_GROUP), :]
        lidx = jnp.min(jnp.where(blk == m, blk_flat_iota, _BIG_I32))
        flat = g * _GROUP_FLAT + lidx
        blk2 = jnp.where(blk_flat_iota == lidx, -jnp.inf, blk)
        fused_ref[pl.ds(g * _ROWS_PER_GROUP, _ROWS_PER_GROUP), :] = blk2
        M = jnp.where(lane_g == g, jnp.max(blk2), M)
        scores_acc = jnp.where(lane == k, m, scores_acc)
        flat_acc = jnp.where(lane == k, flat, flat_acc)
        sel_smem[k] = flat // _NUM_CLASSES
        flat_smem[k] = flat
        return M, scores_acc, flat_acc

    scores0 = jnp.zeros((1, _LANES), dtype=jnp.float32)
    flat0 = jnp.zeros((1, _LANES), dtype=jnp.int32)
    M, scores_acc, flat_acc = jax.lax.fori_loop(
        0, _MAX_DET, topk_body, (M0, scores0, flat0))

    lane_p = jax.lax.broadcasted_iota(jnp.int32, (1, _PACKED), 1)

    def qscore_body(k, scv):
        i = sel_smem[k]
        r2 = i // _NUM_CLASSES
        c2 = i % _NUM_CLASSES
        row = packed_ref[pl.ds(r2, 1), :]
        a = jnp.max(jnp.where(lane_p == c2, row, -jnp.inf))
        b = row[0:1, _NUM_CLASSES:_NUM_CLASSES + 1]
        q = jnp.sqrt(jax.nn.sigmoid(a) * jax.nn.sigmoid(b))
        return jnp.where(lane == k, q, scv)

    scv = jnp.full((1, _LANES), -jnp.inf, dtype=jnp.float32)
    scv = jax.lax.fori_loop(0, _MAX_DET, qscore_body, scv)

    def order_body(k, carry):
        scv, ssorted = carry
        m = jnp.max(scv)
        j = jnp.min(jnp.where(scv == m, lane, _BIG_I32))
        ord_smem[k] = j
        scv = jnp.where(lane == j, -jnp.inf, scv)
        ssorted = jnp.where(lane == k, m, ssorted)
        return scv, ssorted

    ssorted0 = jnp.zeros((1, _LANES), dtype=jnp.float32)
    _, ssorted = jax.lax.fori_loop(0, _MAX_DET, order_body, (scv, ssorted0))

    def gather_body(k, carry):
        x1v, y1v, x2v, y2v, clsv = carry
        j = ord_smem[k]
        i = sel_smem[j]
        ck = flat_smem[j] % _NUM_CLASSES
        row = packed_ref[pl.ds(i, 1), :]
        px = row[0:1, 85:86]
        py = row[0:1, 86:87]
        s = row[0:1, 87:88]
        x1 = px - row[0:1, 81:82] * s
        y1 = py - row[0:1, 82:83] * s
        x2 = px + row[0:1, 83:84] * s
        y2 = py + row[0:1, 84:85] * s
        sel_k = lane == k
        x1v = jnp.where(sel_k, x1, x1v)
        y1v = jnp.where(sel_k, y1, y1v)
        x2v = jnp.where(sel_k, x2, x2v)
        y2v = jnp.where(sel_k, y2, y2v)
        clsv = jnp.where(sel_k, ck, clsv)
        return x1v, y1v, x2v, y2v, clsv

    zeros = jnp.zeros((1, _LANES), dtype=jnp.float32)
    izeros = jnp.zeros((1, _LANES), dtype=jnp.int32)
    x1v, y1v, x2v, y2v, cls_vec = jax.lax.fori_loop(
        0, _MAX_DET, gather_body, (zeros, zeros, zeros, zeros, izeros))

    areav = jnp.maximum(x2v - x1v, 0.0) * jnp.maximum(y2v - y1v, 0.0)

    def nms_body(i, keepf):
        sel_i = lane == i
        keep_i = jnp.max(jnp.where(sel_i, keepf, 0.0)) > 0.0
        xi1 = jnp.max(jnp.where(sel_i, x1v, -jnp.inf))
        yi1 = jnp.max(jnp.where(sel_i, y1v, -jnp.inf))
        xi2 = jnp.max(jnp.where(sel_i, x2v, -jnp.inf))
        yi2 = jnp.max(jnp.where(sel_i, y2v, -jnp.inf))
        ai = jnp.max(jnp.where(sel_i, areav, -jnp.inf))
        iw = jnp.maximum(jnp.minimum(xi2, x2v) - jnp.maximum(xi1, x1v), 0.0)
        ih = jnp.maximum(jnp.minimum(yi2, y2v) - jnp.maximum(yi1, y1v), 0.0)
        inter = iw * ih
        union = ai + areav - inter
        iou = jnp.where(union > 0.0, inter / union, 0.0)
        suppress = keep_i & (iou > _IOU_THR) & (lane > i)
        return jnp.where(suppress, 0.0, keepf)

    keepf = jax.lax.fori_loop(0, _MAX_DET, nms_body,
                              jnp.ones((1, _LANES), dtype=jnp.float32))
    keepv = keepf > 0.0

    boxes_out[0:1, :] = jnp.where(keepv, x1v, 0.0)
    boxes_out[1:2, :] = jnp.where(keepv, y1v, 0.0)
    boxes_out[2:3, :] = jnp.where(keepv, x2v, 0.0)
    boxes_out[3:4, :] = jnp.where(keepv, y2v, 0.0)
    boxes_out[4:8, :] = jnp.zeros((4, _LANES), dtype=jnp.float32)
    scores_out[...] = jnp.where(keepv, ssorted, 0.0)
    classes_out[...] = jnp.where(keepv, cls_vec, -1)


@jax.jit
def kernel(class_preds, box_preds, centerness_preds, points, strides):
    packed = jnp.concatenate(
        [class_preds[0], centerness_preds[0], box_preds[0], points, strides],
        axis=1)

    boxes_r, scores_r, classes_r = pl.pallas_call(
        _fcos_kernel,
        out_shape=[
            jax.ShapeDtypeStruct((8, _LANES), jnp.float32),
            jax.ShapeDtypeStruct((1, _LANES), jnp.float32),
            jax.ShapeDtypeStruct((1, _LANES), jnp.int32),
        ],
        scratch_shapes=[
            pltpu.VMEM((_N, _NUM_CLASSES), jnp.float32),
            pltpu.SMEM((_LANES,), jnp.int32),
            pltpu.SMEM((_LANES,), jnp.int32),
            pltpu.SMEM((_LANES,), jnp.int32),
        ],
    )(packed)

    boxes_out = boxes_r[:4, :_MAX_DET].T
    scores_out = scores_r[0, :_MAX_DET]
    classes_out = classes_r[0, :_MAX_DET]
    return boxes_out, scores_out, classes_out


# speedup vs baseline: 4.4211x; 1.0467x over previous
import functools

import jax
import jax.numpy as jnp
from jax.experimental import pallas as pl
from jax.experimental.pallas import tpu as pltpu

_NUM_CLASSES = 80
_MAX_DET = 100
_IOU_THR = 0.5
_N = 20000
_ROWS_PER_GROUP = 16
_G = _N // _ROWS_PER_GROUP
_GROUP_FLAT = _ROWS_PER_GROUP * _NUM_CLASSES
_LANES = 128

_PACKED = 88
_BIG_I32 = 2 ** 30


def _fcos_kernel(packed_ref, boxes_out, scores_out, classes_out,
                 fused_ref, sel_smem, flat_smem):
    fused = jnp.sqrt(jax.nn.sigmoid(packed_ref[:, 0:_NUM_CLASSES])
                     * jax.nn.sigmoid(packed_ref[:, _NUM_CLASSES:_NUM_CLASSES + 1]))
    fused_ref[...] = fused
    f3 = fused.reshape(_G, _ROWS_PER_GROUP, _NUM_CLASSES)
    gmax = jnp.max(jnp.max(f3, axis=2), axis=1)
    M0 = gmax.reshape(1, _G)

    lane_g = jax.lax.broadcasted_iota(jnp.int32, (1, _G), 1)
    lane = jax.lax.broadcasted_iota(jnp.int32, (1, _LANES), 1)
    blk_flat_iota = (
        jax.lax.broadcasted_iota(jnp.int32, (_ROWS_PER_GROUP, _NUM_CLASSES), 0)
        * _NUM_CLASSES
        + jax.lax.broadcasted_iota(jnp.int32, (_ROWS_PER_GROUP, _NUM_CLASSES), 1)
    )

    lane_p = jax.lax.broadcasted_iota(jnp.int32, (1, _PACKED), 1)

    def topk_body(k, carry):
        M, scv = carry
        m = jnp.max(M)
        g = jnp.min(jnp.where(M == m, lane_g, _BIG_I32))
        blk = fused_ref[pl.ds(g * _ROWS_PER_GROUP, _ROWS_PER_GROUP), :]
        lidx = jnp.min(jnp.where(blk == m, blk_flat_iota, _BIG_I32))
        flat = g * _GROUP_FLAT + lidx
        blk2 = jnp.where(blk_flat_iota == lidx, -jnp.inf, blk)
        fused_ref[pl.ds(g * _ROWS_PER_GROUP, _ROWS_PER_GROUP), :] = blk2
        M = jnp.where(lane_g == g, jnp.max(blk2), M)
        i = flat // _NUM_CLASSES
        sel_smem[k] = i
        flat_smem[k] = flat
        r2 = i // _NUM_CLASSES
        c2 = i % _NUM_CLASSES
        row = packed_ref[pl.ds(r2, 1), :]
        a = jnp.max(jnp.where(lane_p == c2, row, -jnp.inf))
        b = row[0:1, _NUM_CLASSES:_NUM_CLASSES + 1]
        q = jnp.sqrt(jax.nn.sigmoid(a) * jax.nn.sigmoid(b))
        scv = jnp.where(lane == k, q, scv)
        return M, scv

    scv0 = jnp.full((1, _LANES), -jnp.inf, dtype=jnp.float32)
    M, scv = jax.lax.fori_loop(0, _MAX_DET, topk_body, (M0, scv0))

    def order_body(k, carry):
        scv, ssorted, x1v, y1v, x2v, y2v, clsv = carry
        m = jnp.max(scv)
        j = jnp.min(jnp.where(scv == m, lane, _BIG_I32))
        scv = jnp.where(lane == j, -jnp.inf, scv)
        ssorted = jnp.where(lane == k, m, ssorted)
        i = sel_smem[j]
        ck = flat_smem[j] % _NUM_CLASSES
        row = packed_ref[pl.ds(i, 1), :]
        px = row[0:1, 85:86]
        py = row[0:1, 86:87]
        s = row[0:1, 87:88]
        x1 = px - row[0:1, 81:82] * s
        y1 = py - row[0:1, 82:83] * s
        x2 = px + row[0:1, 83:84] * s
        y2 = py + row[0:1, 84:85] * s
        sel_k = lane == k
        x1v = jnp.where(sel_k, x1, x1v)
        y1v = jnp.where(sel_k, y1, y1v)
        x2v = jnp.where(sel_k, x2, x2v)
        y2v = jnp.where(sel_k, y2, y2v)
        clsv = jnp.where(sel_k, ck, clsv)
        return scv, ssorted, x1v, y1v, x2v, y2v, clsv

    zeros = jnp.zeros((1, _LANES), dtype=jnp.float32)
    izeros = jnp.zeros((1, _LANES), dtype=jnp.int32)
    _, ssorted, x1v, y1v, x2v, y2v, cls_vec = jax.lax.fori_loop(
        0, _MAX_DET, order_body,
        (scv, zeros, zeros, zeros, zeros, zeros, izeros))

    areav = jnp.maximum(x2v - x1v, 0.0) * jnp.maximum(y2v - y1v, 0.0)

    def nms_body(i, keepf):
        sel_i = lane == i
        keep_i = jnp.max(jnp.where(sel_i, keepf, 0.0)) > 0.0
        xi1 = jnp.max(jnp.where(sel_i, x1v, -jnp.inf))
        yi1 = jnp.max(jnp.where(sel_i, y1v, -jnp.inf))
        xi2 = jnp.max(jnp.where(sel_i, x2v, -jnp.inf))
        yi2 = jnp.max(jnp.where(sel_i, y2v, -jnp.inf))
        ai = jnp.max(jnp.where(sel_i, areav, -jnp.inf))
        iw = jnp.maximum(jnp.minimum(xi2, x2v) - jnp.maximum(xi1, x1v), 0.0)
        ih = jnp.maximum(jnp.minimum(yi2, y2v) - jnp.maximum(yi1, y1v), 0.0)
        inter = iw * ih
        union = ai + areav - inter
        iou = jnp.where(union > 0.0, inter / union, 0.0)
        suppress = keep_i & (iou > _IOU_THR) & (lane > i)
        return jnp.where(suppress, 0.0, keepf)

    keepf = jax.lax.fori_loop(0, _MAX_DET, nms_body,
                              jnp.ones((1, _LANES), dtype=jnp.float32))
    keepv = keepf > 0.0

    boxes_out[0:1, :] = jnp.where(keepv, x1v, 0.0)
    boxes_out[1:2, :] = jnp.where(keepv, y1v, 0.0)
    boxes_out[2:3, :] = jnp.where(keepv, x2v, 0.0)
    boxes_out[3:4, :] = jnp.where(keepv, y2v, 0.0)
    boxes_out[4:8, :] = jnp.zeros((4, _LANES), dtype=jnp.float32)
    scores_out[...] = jnp.where(keepv, ssorted, 0.0)
    classes_out[...] = jnp.where(keepv, cls_vec, -1)


@jax.jit
def kernel(class_preds, box_preds, centerness_preds, points, strides):
    packed = jnp.concatenate(
        [class_preds[0], centerness_preds[0], box_preds[0], points, strides],
        axis=1)

    boxes_r, scores_r, classes_r = pl.pallas_call(
        _fcos_kernel,
        out_shape=[
            jax.ShapeDtypeStruct((8, _LANES), jnp.float32),
            jax.ShapeDtypeStruct((1, _LANES), jnp.float32),
            jax.ShapeDtypeStruct((1, _LANES), jnp.int32),
        ],
        scratch_shapes=[
            pltpu.VMEM((_N, _NUM_CLASSES), jnp.float32),
            pltpu.SMEM((_LANES,), jnp.int32),
            pltpu.SMEM((_LANES,), jnp.int32),
        ],
    )(packed)

    boxes_out = boxes_r[:4, :_MAX_DET].T
    scores_out = scores_r[0, :_MAX_DET]
    classes_out = classes_r[0, :_MAX_DET]
    return boxes_out, scores_out, classes_out


# speedup vs baseline: 5.5889x; 1.2642x over previous
import jax
import jax.numpy as jnp
from jax.experimental import pallas as pl
from jax.experimental.pallas import tpu as pltpu

_NUM_CLASSES = 80
_MAX_DET = 100
_IOU_THR = 0.5
_N = 20000
_ROWS_PER_GROUP = 16
_G = _N // _ROWS_PER_GROUP
_GROUP_FLAT = _ROWS_PER_GROUP * _NUM_CLASSES
_LANES = 128

_PACKED = 88
_BIG_I32 = 2 ** 30


def _fcos_kernel(packed_ref, boxes_out, scores_out, classes_out,
                 fused_ref, iou_ref):
    fused = jnp.sqrt(jax.nn.sigmoid(packed_ref[:, 0:_NUM_CLASSES])
                     * jax.nn.sigmoid(packed_ref[:, _NUM_CLASSES:_NUM_CLASSES + 1]))
    fused_ref[...] = fused
    f3 = fused.reshape(_G, _ROWS_PER_GROUP, _NUM_CLASSES)
    gmax = jnp.max(jnp.max(f3, axis=2), axis=1)
    M0 = gmax.reshape(1, _G)

    lane_g = jax.lax.broadcasted_iota(jnp.int32, (1, _G), 1)
    lane = jax.lax.broadcasted_iota(jnp.int32, (1, _LANES), 1)
    lane_p = jax.lax.broadcasted_iota(jnp.int32, (1, _PACKED), 1)
    blk_flat_iota = (
        jax.lax.broadcasted_iota(jnp.int32, (_ROWS_PER_GROUP, _NUM_CLASSES), 0)
        * _NUM_CLASSES
        + jax.lax.broadcasted_iota(jnp.int32, (_ROWS_PER_GROUP, _NUM_CLASSES), 1)
    )

    def topk_body(k, carry):
        M, scv, x1v, y1v, x2v, y2v, clsv = carry
        m = jnp.max(M)
        g = jnp.min(jnp.where(M == m, lane_g, _BIG_I32))
        blk = fused_ref[pl.ds(g * _ROWS_PER_GROUP, _ROWS_PER_GROUP), :]
        lidx = jnp.min(jnp.where(blk == m, blk_flat_iota, _BIG_I32))
        flat = g * _GROUP_FLAT + lidx
        blk2 = jnp.where(blk_flat_iota == lidx, -jnp.inf, blk)
        fused_ref[pl.ds(g * _ROWS_PER_GROUP, _ROWS_PER_GROUP), :] = blk2
        M = jnp.where(lane_g == g, jnp.max(blk2), M)
        i = flat // _NUM_CLASSES
        ck = flat % _NUM_CLASSES
        r2 = i // _NUM_CLASSES
        c2 = i % _NUM_CLASSES
        qrow = packed_ref[pl.ds(r2, 1), :]
        a = jnp.max(jnp.where(lane_p == c2, qrow, -jnp.inf))
        b = qrow[0:1, _NUM_CLASSES:_NUM_CLASSES + 1]
        q = jnp.sqrt(jax.nn.sigmoid(a) * jax.nn.sigmoid(b))
        row = packed_ref[pl.ds(i, 1), :]
        px = row[0:1, 85:86]
        py = row[0:1, 86:87]
        s = row[0:1, 87:88]
        sel_k = lane == k
        scv = jnp.where(sel_k, q, scv)
        x1v = jnp.where(sel_k, px - row[0:1, 81:82] * s, x1v)
        y1v = jnp.where(sel_k, py - row[0:1, 82:83] * s, y1v)
        x2v = jnp.where(sel_k, px + row[0:1, 83:84] * s, x2v)
        y2v = jnp.where(sel_k, py + row[0:1, 84:85] * s, y2v)
        clsv = jnp.where(sel_k, ck, clsv)
        return M, scv, x1v, y1v, x2v, y2v, clsv

    zeros = jnp.zeros((1, _LANES), dtype=jnp.float32)
    izeros = jnp.zeros((1, _LANES), dtype=jnp.int32)
    scv0 = jnp.full((1, _LANES), -jnp.inf, dtype=jnp.float32)
    _, scv, x1v, y1v, x2v, y2v, clsv = jax.lax.fori_loop(
        0, _MAX_DET, topk_body,
        (M0, scv0, zeros, zeros, zeros, zeros, izeros))

    sub_col = jax.lax.broadcasted_iota(jnp.int32, (_LANES, 1), 0)
    lane_f = lane.astype(jnp.float32)

    def to_col(rowvec):
        w = jnp.where(lane == sub_col, rowvec, -jnp.inf)
        return jnp.max(w, axis=1, keepdims=True)

    s_col = to_col(scv)
    pre = (s_col > scv) | ((s_col == scv) & (sub_col < lane))
    rank = jnp.sum(pre.astype(jnp.float32), axis=0, keepdims=True)
    rank_col = to_col(rank)
    perm_t = (rank_col == lane_f).astype(jnp.float32)

    feats = jnp.concatenate(
        [x1v, y1v, x2v, y2v, jnp.maximum(scv, 0.0), clsv.astype(jnp.float32),
         jnp.zeros((2, _LANES), dtype=jnp.float32)], axis=0)
    sorted_f = jnp.dot(feats, perm_t,
                       precision=jax.lax.Precision.HIGHEST,
                       preferred_element_type=jnp.float32)
    x1s = sorted_f[0:1, :]
    y1s = sorted_f[1:2, :]
    x2s = sorted_f[2:3, :]
    y2s = sorted_f[3:4, :]
    qs = sorted_f[4:5, :]
    clss = sorted_f[5:6, :].astype(jnp.int32)

    area = jnp.maximum(x2s - x1s, 0.0) * jnp.maximum(y2s - y1s, 0.0)
    x1c = to_col(x1s)
    y1c = to_col(y1s)
    x2c = to_col(x2s)
    y2c = to_col(y2s)
    area_c = to_col(area)
    iw = jnp.maximum(jnp.minimum(x2c, x2s) - jnp.maximum(x1c, x1s), 0.0)
    ih = jnp.maximum(jnp.minimum(y2c, y2s) - jnp.maximum(y1c, y1s), 0.0)
    inter = iw * ih
    union = area_c + area - inter
    iou_ref[...] = jnp.where(union > 0.0, inter / union, 0.0)

    def nms_body(i, keepf):
        irow = iou_ref[pl.ds(i, 1), :]
        keep_i = jnp.max(jnp.where(lane == i, keepf, 0.0)) > 0.0
        suppress = keep_i & (irow > _IOU_THR) & (lane > i)
        return jnp.where(suppress, 0.0, keepf)

    keepf = jax.lax.fori_loop(0, _MAX_DET, nms_body,
                              jnp.ones((1, _LANES), dtype=jnp.float32))
    keepv = keepf > 0.0

    boxes_out[0:1, :] = jnp.where(keepv, x1s, 0.0)
    boxes_out[1:2, :] = jnp.where(keepv, y1s, 0.0)
    boxes_out[2:3, :] = jnp.where(keepv, x2s, 0.0)
    boxes_out[3:4, :] = jnp.where(keepv, y2s, 0.0)
    boxes_out[4:8, :] = jnp.zeros((4, _LANES), dtype=jnp.float32)
    scores_out[...] = jnp.where(keepv, qs, 0.0)
    classes_out[...] = jnp.where(keepv, clss, -1)


@jax.jit
def kernel(class_preds, box_preds, centerness_preds, points, strides):
    packed = jnp.concatenate(
        [class_preds[0], centerness_preds[0], box_preds[0], points, strides],
        axis=1)

    boxes_r, scores_r, classes_r = pl.pallas_call(
        _fcos_kernel,
        out_shape=[
            jax.ShapeDtypeStruct((8, _LANES), jnp.float32),
            jax.ShapeDtypeStruct((1, _LANES), jnp.float32),
            jax.ShapeDtypeStruct((1, _LANES), jnp.int32),
        ],
        scratch_shapes=[
            pltpu.VMEM((_N, _NUM_CLASSES), jnp.float32),
            pltpu.VMEM((_LANES, _LANES), jnp.float32),
        ],
    )(packed)

    boxes_out = boxes_r[:4, :_MAX_DET].T
    scores_out = scores_r[0, :_MAX_DET]
    classes_out = classes_r[0, :_MAX_DET]
    return boxes_out, scores_out, classes_out


# speedup vs baseline: 6.5568x; 1.1732x over previous
import jax
import jax.numpy as jnp
from jax.experimental import pallas as pl
from jax.experimental.pallas import tpu as pltpu

_NUM_CLASSES = 80
_MAX_DET = 100
_IOU_THR = 0.5
_N = 20000
_ROWS_PER_GROUP = 16
_G = _N // _ROWS_PER_GROUP
_GROUP_FLAT = _ROWS_PER_GROUP * _NUM_CLASSES
_LANES = 128

_BIG_I32 = 2 ** 30


def _fcos_kernel(cls_ref, small_ref, boxes_out, scores_out, classes_out,
                 fused_ref, iou_ref):
    fused = jnp.sqrt(jax.nn.sigmoid(cls_ref[...])
                     * jax.nn.sigmoid(small_ref[:, 0:1]))
    fused_ref[...] = fused
    f3 = fused.reshape(_G, _ROWS_PER_GROUP, _NUM_CLASSES)
    gmax = jnp.max(jnp.max(f3, axis=2), axis=1)
    M0 = gmax.reshape(1, _G)

    lane_g = jax.lax.broadcasted_iota(jnp.int32, (1, _G), 1)
    lane = jax.lax.broadcasted_iota(jnp.int32, (1, _LANES), 1)
    lane_c = jax.lax.broadcasted_iota(jnp.int32, (1, _NUM_CLASSES), 1)
    blk_flat_iota = (
        jax.lax.broadcasted_iota(jnp.int32, (_ROWS_PER_GROUP, _NUM_CLASSES), 0)
        * _NUM_CLASSES
        + jax.lax.broadcasted_iota(jnp.int32, (_ROWS_PER_GROUP, _NUM_CLASSES), 1)
    )

    def topk_body(k, carry):
        M, scv, x1v, y1v, x2v, y2v, clsv = carry
        m = jnp.max(M)
        g = jnp.min(jnp.where(M == m, lane_g, _BIG_I32))
        blk = fused_ref[pl.ds(g * _ROWS_PER_GROUP, _ROWS_PER_GROUP), :]
        lidx = jnp.min(jnp.where(blk == m, blk_flat_iota, _BIG_I32))
        flat = g * _GROUP_FLAT + lidx
        blk2 = jnp.where(blk_flat_iota == lidx, -jnp.inf, blk)
        fused_ref[pl.ds(g * _ROWS_PER_GROUP, _ROWS_PER_GROUP), :] = blk2
        M = jnp.where(lane_g == g, jnp.max(blk2), M)
        i = flat // _NUM_CLASSES
        ck = flat % _NUM_CLASSES
        r2 = i // _NUM_CLASSES
        c2 = i % _NUM_CLASSES
        qrow = cls_ref[pl.ds(r2, 1), :]
        a = jnp.max(jnp.where(lane_c == c2, qrow, -jnp.inf))
        b = small_ref[pl.ds(r2, 1), 0:1]
        q = jnp.sqrt(jax.nn.sigmoid(a) * jax.nn.sigmoid(b))
        row = small_ref[pl.ds(i, 1), :]
        px = row[0:1, 5:6]
        py = row[0:1, 6:7]
        s = row[0:1, 7:8]
        sel_k = lane == k
        scv = jnp.where(sel_k, q, scv)
        x1v = jnp.where(sel_k, px - row[0:1, 1:2] * s, x1v)
        y1v = jnp.where(sel_k, py - row[0:1, 2:3] * s, y1v)
        x2v = jnp.where(sel_k, px + row[0:1, 3:4] * s, x2v)
        y2v = jnp.where(sel_k, py + row[0:1, 4:5] * s, y2v)
        clsv = jnp.where(sel_k, ck, clsv)
        return M, scv, x1v, y1v, x2v, y2v, clsv

    zeros = jnp.zeros((1, _LANES), dtype=jnp.float32)
    izeros = jnp.zeros((1, _LANES), dtype=jnp.int32)
    scv0 = jnp.full((1, _LANES), -jnp.inf, dtype=jnp.float32)
    _, scv, x1v, y1v, x2v, y2v, clsv = jax.lax.fori_loop(
        0, _MAX_DET, topk_body,
        (M0, scv0, zeros, zeros, zeros, zeros, izeros))

    sub_col = jax.lax.broadcasted_iota(jnp.int32, (_LANES, 1), 0)
    lane_f = lane.astype(jnp.float32)

    def to_col(rowvec):
        w = jnp.where(lane == sub_col, rowvec, -jnp.inf)
        return jnp.max(w, axis=1, keepdims=True)

    s_col = to_col(scv)
    pre = (s_col > scv) | ((s_col == scv) & (sub_col < lane))
    rank = jnp.sum(pre.astype(jnp.float32), axis=0, keepdims=True)
    rank_col = to_col(rank)
    perm_t = (rank_col == lane_f).astype(jnp.float32)

    feats = jnp.concatenate(
        [x1v, y1v, x2v, y2v, jnp.maximum(scv, 0.0), clsv.astype(jnp.float32),
         jnp.zeros((2, _LANES), dtype=jnp.float32)], axis=0)
    sorted_f = jnp.dot(feats, perm_t,
                       precision=jax.lax.Precision.HIGHEST,
                       preferred_element_type=jnp.float32)
    x1s = sorted_f[0:1, :]
    y1s = sorted_f[1:2, :]
    x2s = sorted_f[2:3, :]
    y2s = sorted_f[3:4, :]
    qs = sorted_f[4:5, :]
    clss = sorted_f[5:6, :].astype(jnp.int32)

    area = jnp.maximum(x2s - x1s, 0.0) * jnp.maximum(y2s - y1s, 0.0)
    x1c = to_col(x1s)
    y1c = to_col(y1s)
    x2c = to_col(x2s)
    y2c = to_col(y2s)
    area_c = to_col(area)
    iw = jnp.maximum(jnp.minimum(x2c, x2s) - jnp.maximum(x1c, x1s), 0.0)
    ih = jnp.maximum(jnp.minimum(y2c, y2s) - jnp.maximum(y1c, y1s), 0.0)
    inter = iw * ih
    union = area_c + area - inter
    iou_ref[...] = jnp.where(union > 0.0, inter / union, 0.0)

    def nms_body(i, keepf):
        irow = iou_ref[pl.ds(i, 1), :]
        keep_i = jnp.max(jnp.where(lane == i, keepf, 0.0)) > 0.0
        suppress = keep_i & (irow > _IOU_THR) & (lane > i)
        return jnp.where(suppress, 0.0, keepf)

    keepf = jax.lax.fori_loop(0, _MAX_DET, nms_body,
                              jnp.ones((1, _LANES), dtype=jnp.float32))
    keepv = keepf > 0.0

    boxes_out[0:1, :] = jnp.where(keepv, x1s, 0.0)
    boxes_out[1:2, :] = jnp.where(keepv, y1s, 0.0)
    boxes_out[2:3, :] = jnp.where(keepv, x2s, 0.0)
    boxes_out[3:4, :] = jnp.where(keepv, y2s, 0.0)
    boxes_out[4:8, :] = jnp.zeros((4, _LANES), dtype=jnp.float32)
    scores_out[...] = jnp.where(keepv, qs, 0.0)
    classes_out[...] = jnp.where(keepv, clss, -1)


@jax.jit
def kernel(class_preds, box_preds, centerness_preds, points, strides):
    small = jnp.concatenate(
        [centerness_preds[0], box_preds[0], points, strides],
        axis=1)

    boxes_r, scores_r, classes_r = pl.pallas_call(
        _fcos_kernel,
        out_shape=[
            jax.ShapeDtypeStruct((8, _LANES), jnp.float32),
            jax.ShapeDtypeStruct((1, _LANES), jnp.float32),
            jax.ShapeDtypeStruct((1, _LANES), jnp.int32),
        ],
        scratch_shapes=[
            pltpu.VMEM((_N, _NUM_CLASSES), jnp.float32),
            pltpu.VMEM((_LANES, _LANES), jnp.float32),
        ],
    )(class_preds[0], small)

    boxes_out = boxes_r[:4, :_MAX_DET].T
    scores_out = scores_r[0, :_MAX_DET]
    classes_out = classes_r[0, :_MAX_DET]
    return boxes_out, scores_out, classes_out


# speedup vs baseline: 7.2247x; 1.1019x over previous
import jax
import jax.numpy as jnp
from jax.experimental import pallas as pl
from jax.experimental.pallas import tpu as pltpu

_NUM_CLASSES = 80
_MAX_DET = 100
_IOU_THR = 0.5
_N = 20000
_ROWS_PER_GROUP = 16
_G = _N // _ROWS_PER_GROUP
_GROUP_FLAT = _ROWS_PER_GROUP * _NUM_CLASSES
_LANES = 128

_BIG_I32 = 2 ** 30


def _fcos_kernel(cls_ref, small_ref, boxes_out, scores_out, classes_out,
                 fused_ref, iou_ref):
    fused = jnp.sqrt(jax.nn.sigmoid(cls_ref[...])
                     * jax.nn.sigmoid(small_ref[:, 0:1]))
    fused_ref[...] = fused
    f3 = fused.reshape(_G, _ROWS_PER_GROUP, _NUM_CLASSES)
    gmax = jnp.max(jnp.max(f3, axis=2), axis=1)
    M0 = gmax.reshape(1, _G)

    lane_g = jax.lax.broadcasted_iota(jnp.int32, (1, _G), 1)
    lane = jax.lax.broadcasted_iota(jnp.int32, (1, _LANES), 1)
    lane_c = jax.lax.broadcasted_iota(jnp.int32, (1, _NUM_CLASSES), 1)
    blk_flat_iota = (
        jax.lax.broadcasted_iota(jnp.int32, (_ROWS_PER_GROUP, _NUM_CLASSES), 0)
        * _NUM_CLASSES
        + jax.lax.broadcasted_iota(jnp.int32, (_ROWS_PER_GROUP, _NUM_CLASSES), 1)
    )

    def topk_body(k, carry):
        M, scv, x1v, y1v, x2v, y2v, clsv = carry
        mm = jnp.max(M, axis=1, keepdims=True)
        g = jnp.min(jnp.where(M == mm, lane_g, _BIG_I32))
        blk = fused_ref[pl.ds(g * _ROWS_PER_GROUP, _ROWS_PER_GROUP), :]
        lidx = jnp.min(jnp.where(blk == mm, blk_flat_iota, _BIG_I32))
        flat = g * _GROUP_FLAT + lidx
        blk2 = jnp.where(blk_flat_iota == lidx, -jnp.inf, blk)
        fused_ref[pl.ds(g * _ROWS_PER_GROUP, _ROWS_PER_GROUP), :] = blk2
        bm = jnp.max(jnp.max(blk2, axis=0, keepdims=True), axis=1, keepdims=True)
        M = jnp.where(lane_g == g, bm, M)
        i = flat // _NUM_CLASSES
        ck = flat % _NUM_CLASSES
        r2 = i // _NUM_CLASSES
        c2 = i % _NUM_CLASSES
        qrow = cls_ref[pl.ds(r2, 1), :]
        a = jnp.max(jnp.where(lane_c == c2, qrow, -jnp.inf), axis=1, keepdims=True)
        b = small_ref[pl.ds(r2, 1), 0:1]
        q = jnp.sqrt(jax.nn.sigmoid(a) * jax.nn.sigmoid(b))
        row = small_ref[pl.ds(i, 1), :]
        px = row[0:1, 5:6]
        py = row[0:1, 6:7]
        s = row[0:1, 7:8]
        sel_k = lane == k
        scv = jnp.where(sel_k, q, scv)
        x1v = jnp.where(sel_k, px - row[0:1, 1:2] * s, x1v)
        y1v = jnp.where(sel_k, py - row[0:1, 2:3] * s, y1v)
        x2v = jnp.where(sel_k, px + row[0:1, 3:4] * s, x2v)
        y2v = jnp.where(sel_k, py + row[0:1, 4:5] * s, y2v)
        clsv = jnp.where(sel_k, ck, clsv)
        return M, scv, x1v, y1v, x2v, y2v, clsv

    zeros = jnp.zeros((1, _LANES), dtype=jnp.float32)
    izeros = jnp.zeros((1, _LANES), dtype=jnp.int32)
    scv0 = jnp.full((1, _LANES), -jnp.inf, dtype=jnp.float32)
    _, scv, x1v, y1v, x2v, y2v, clsv = jax.lax.fori_loop(
        0, _MAX_DET, topk_body,
        (M0, scv0, zeros, zeros, zeros, zeros, izeros))

    sub_col = jax.lax.broadcasted_iota(jnp.int32, (_LANES, 1), 0)
    lane_f = lane.astype(jnp.float32)

    def to_col(rowvec):
        w = jnp.where(lane == sub_col, rowvec, -jnp.inf)
        return jnp.max(w, axis=1, keepdims=True)

    s_col = to_col(scv)
    pre = (s_col > scv) | ((s_col == scv) & (sub_col < lane))
    rank = jnp.sum(pre.astype(jnp.float32), axis=0, keepdims=True)
    rank_col = to_col(rank)
    perm_t = (rank_col == lane_f).astype(jnp.float32)

    feats = jnp.concatenate(
        [x1v, y1v, x2v, y2v, jnp.maximum(scv, 0.0), clsv.astype(jnp.float32),
         jnp.zeros((2, _LANES), dtype=jnp.float32)], axis=0)
    sorted_f = jnp.dot(feats, perm_t,
                       precision=jax.lax.Precision.HIGHEST,
                       preferred_element_type=jnp.float32)
    x1s = sorted_f[0:1, :]
    y1s = sorted_f[1:2, :]
    x2s = sorted_f[2:3, :]
    y2s = sorted_f[3:4, :]
    qs = sorted_f[4:5, :]
    clss = sorted_f[5:6, :].astype(jnp.int32)

    area = jnp.maximum(x2s - x1s, 0.0) * jnp.maximum(y2s - y1s, 0.0)
    x1c = to_col(x1s)
    y1c = to_col(y1s)
    x2c = to_col(x2s)
    y2c = to_col(y2s)
    area_c = to_col(area)
    iw = jnp.maximum(jnp.minimum(x2c, x2s) - jnp.maximum(x1c, x1s), 0.0)
    ih = jnp.maximum(jnp.minimum(y2c, y2s) - jnp.maximum(y1c, y1s), 0.0)
    inter = iw * ih
    union = area_c + area - inter
    iou_ref[...] = jnp.where(union > 0.0, inter / union, 0.0)

    def nms_body(i, keepf):
        irow = iou_ref[pl.ds(i, 1), :]
        keep_i = jnp.max(jnp.where(lane == i, keepf, 0.0), axis=1, keepdims=True) > 0.0
        suppress = keep_i & (irow > _IOU_THR) & (lane > i)
        return jnp.where(suppress, 0.0, keepf)

    keepf = jax.lax.fori_loop(0, _MAX_DET, nms_body,
                              jnp.ones((1, _LANES), dtype=jnp.float32))
    keepv = keepf > 0.0

    boxes_out[0:1, :] = jnp.where(keepv, x1s, 0.0)
    boxes_out[1:2, :] = jnp.where(keepv, y1s, 0.0)
    boxes_out[2:3, :] = jnp.where(keepv, x2s, 0.0)
    boxes_out[3:4, :] = jnp.where(keepv, y2s, 0.0)
    boxes_out[4:8, :] = jnp.zeros((4, _LANES), dtype=jnp.float32)
    scores_out[...] = jnp.where(keepv, qs, 0.0)
    classes_out[...] = jnp.where(keepv, clss, -1)


@jax.jit
def kernel(class_preds, box_preds, centerness_preds, points, strides):
    small = jnp.concatenate(
        [centerness_preds[0], box_preds[0], points, strides],
        axis=1)

    boxes_r, scores_r, classes_r = pl.pallas_call(
        _fcos_kernel,
        out_shape=[
            jax.ShapeDtypeStruct((8, _LANES), jnp.float32),
            jax.ShapeDtypeStruct((1, _LANES), jnp.float32),
            jax.ShapeDtypeStruct((1, _LANES), jnp.int32),
        ],
        scratch_shapes=[
            pltpu.VMEM((_N, _NUM_CLASSES), jnp.float32),
            pltpu.VMEM((_LANES, _LANES), jnp.float32),
        ],
    )(class_preds[0], small)

    boxes_out = boxes_r[:4, :_MAX_DET].T
    scores_out = scores_r[0, :_MAX_DET]
    classes_out = classes_r[0, :_MAX_DET]
    return boxes_out, scores_out, classes_out


# speedup vs baseline: 7.2947x; 1.0097x over previous
import jax
import jax.numpy as jnp
from jax.experimental import pallas as pl
from jax.experimental.pallas import tpu as pltpu

_NUM_CLASSES = 80
_MAX_DET = 100
_IOU_THR = 0.5
_N = 20000
_ROWS_PER_GROUP = 16
_G = _N // _ROWS_PER_GROUP
_GROUP_FLAT = _ROWS_PER_GROUP * _NUM_CLASSES
_LANES = 128

_BIG_I32 = 2 ** 30


def _fcos_kernel(cls_ref, small_ref, boxes_out, scores_out, classes_out,
                 fused_ref, iou_ref):
    fused = jax.nn.sigmoid(cls_ref[...]) * jax.nn.sigmoid(small_ref[:, 0:1])
    fused_ref[...] = fused
    f3 = fused.reshape(_G, _ROWS_PER_GROUP, _NUM_CLASSES)
    gmax = jnp.max(jnp.max(f3, axis=2), axis=1)
    M0 = gmax.reshape(1, _G)

    lane_g = jax.lax.broadcasted_iota(jnp.int32, (1, _G), 1)
    lane = jax.lax.broadcasted_iota(jnp.int32, (1, _LANES), 1)
    lane_c = jax.lax.broadcasted_iota(jnp.int32, (1, _NUM_CLASSES), 1)
    blk_flat_iota = (
        jax.lax.broadcasted_iota(jnp.int32, (_ROWS_PER_GROUP, _NUM_CLASSES), 0)
        * _NUM_CLASSES
        + jax.lax.broadcasted_iota(jnp.int32, (_ROWS_PER_GROUP, _NUM_CLASSES), 1)
    )

    def topk_body(k, carry):
        M, scv, x1v, y1v, x2v, y2v, clsv = carry
        mm = jnp.max(M, axis=1, keepdims=True)
        g = jnp.min(jnp.where(M == mm, lane_g, _BIG_I32))
        blk = fused_ref[pl.ds(g * _ROWS_PER_GROUP, _ROWS_PER_GROUP), :]
        lidx = jnp.min(jnp.where(blk == mm, blk_flat_iota, _BIG_I32))
        flat = g * _GROUP_FLAT + lidx
        blk2 = jnp.where(blk_flat_iota == lidx, -jnp.inf, blk)
        fused_ref[pl.ds(g * _ROWS_PER_GROUP, _ROWS_PER_GROUP), :] = blk2
        bm = jnp.max(jnp.max(blk2, axis=0, keepdims=True), axis=1, keepdims=True)
        M = jnp.where(lane_g == g, bm, M)
        i = flat // _NUM_CLASSES
        ck = flat % _NUM_CLASSES
        r2 = i // _NUM_CLASSES
        c2 = i % _NUM_CLASSES
        qrow = cls_ref[pl.ds(r2, 1), :]
        a = jnp.max(jnp.where(lane_c == c2, qrow, -jnp.inf), axis=1, keepdims=True)
        b = small_ref[pl.ds(r2, 1), 0:1]
        q = jnp.sqrt(jax.nn.sigmoid(a) * jax.nn.sigmoid(b))
        row = small_ref[pl.ds(i, 1), :]
        px = row[0:1, 5:6]
        py = row[0:1, 6:7]
        s = row[0:1, 7:8]
        sel_k = lane == k
        scv = jnp.where(sel_k, q, scv)
        x1v = jnp.where(sel_k, px - row[0:1, 1:2] * s, x1v)
        y1v = jnp.where(sel_k, py - row[0:1, 2:3] * s, y1v)
        x2v = jnp.where(sel_k, px + row[0:1, 3:4] * s, x2v)
        y2v = jnp.where(sel_k, py + row[0:1, 4:5] * s, y2v)
        clsv = jnp.where(sel_k, ck, clsv)
        return M, scv, x1v, y1v, x2v, y2v, clsv

    zeros = jnp.zeros((1, _LANES), dtype=jnp.float32)
    izeros = jnp.zeros((1, _LANES), dtype=jnp.int32)
    scv0 = jnp.full((1, _LANES), -jnp.inf, dtype=jnp.float32)
    _, scv, x1v, y1v, x2v, y2v, clsv = jax.lax.fori_loop(
        0, _MAX_DET, topk_body,
        (M0, scv0, zeros, zeros, zeros, zeros, izeros))

    sub_col = jax.lax.broadcasted_iota(jnp.int32, (_LANES, 1), 0)
    lane_f = lane.astype(jnp.float32)

    def to_col(rowvec):
        w = jnp.where(lane == sub_col, rowvec, -jnp.inf)
        return jnp.max(w, axis=1, keepdims=True)

    s_col = to_col(scv)
    pre = (s_col > scv) | ((s_col == scv) & (sub_col < lane))
    rank = jnp.sum(pre.astype(jnp.float32), axis=0, keepdims=True)
    rank_col = to_col(rank)
    perm_t = (rank_col == lane_f).astype(jnp.float32)

    feats = jnp.concatenate(
        [x1v, y1v, x2v, y2v, jnp.maximum(scv, 0.0), clsv.astype(jnp.float32),
         jnp.zeros((2, _LANES), dtype=jnp.float32)], axis=0)
    sorted_f = jnp.dot(feats, perm_t,
                       precision=jax.lax.Precision.HIGHEST,
                       preferred_element_type=jnp.float32)
    x1s = sorted_f[0:1, :]
    y1s = sorted_f[1:2, :]
    x2s = sorted_f[2:3, :]
    y2s = sorted_f[3:4, :]
    qs = sorted_f[4:5, :]
    clss = sorted_f[5:6, :].astype(jnp.int32)

    area = jnp.maximum(x2s - x1s, 0.0) * jnp.maximum(y2s - y1s, 0.0)
    x1c = to_col(x1s)
    y1c = to_col(y1s)
    x2c = to_col(x2s)
    y2c = to_col(y2s)
    area_c = to_col(area)
    iw = jnp.maximum(jnp.minimum(x2c, x2s) - jnp.maximum(x1c, x1s), 0.0)
    ih = jnp.maximum(jnp.minimum(y2c, y2s) - jnp.maximum(y1c, y1s), 0.0)
    inter = iw * ih
    union = area_c + area - inter
    iou_ref[...] = jnp.where(union > 0.0, inter / union, 0.0)

    def nms_body(i, keepf):
        irow = iou_ref[pl.ds(i, 1), :]
        keep_i = jnp.max(jnp.where(lane == i, keepf, 0.0), axis=1, keepdims=True) > 0.0
        suppress = keep_i & (irow > _IOU_THR) & (lane > i)
        return jnp.where(suppress, 0.0, keepf)

    keepf = jax.lax.fori_loop(0, _MAX_DET, nms_body,
                              jnp.ones((1, _LANES), dtype=jnp.float32))
    keepv = keepf > 0.0

    boxes_out[0:1, :] = jnp.where(keepv, x1s, 0.0)
    boxes_out[1:2, :] = jnp.where(keepv, y1s, 0.0)
    boxes_out[2:3, :] = jnp.where(keepv, x2s, 0.0)
    boxes_out[3:4, :] = jnp.where(keepv, y2s, 0.0)
    boxes_out[4:8, :] = jnp.zeros((4, _LANES), dtype=jnp.float32)
    scores_out[...] = jnp.where(keepv, qs, 0.0)
    classes_out[...] = jnp.where(keepv, clss, -1)


@jax.jit
def kernel(class_preds, box_preds, centerness_preds, points, strides):
    small = jnp.concatenate(
        [centerness_preds[0], box_preds[0], points, strides],
        axis=1)

    boxes_r, scores_r, classes_r = pl.pallas_call(
        _fcos_kernel,
        out_shape=[
            jax.ShapeDtypeStruct((8, _LANES), jnp.float32),
            jax.ShapeDtypeStruct((1, _LANES), jnp.float32),
            jax.ShapeDtypeStruct((1, _LANES), jnp.int32),
        ],
        scratch_shapes=[
            pltpu.VMEM((_N, _NUM_CLASSES), jnp.float32),
            pltpu.VMEM((_LANES, _LANES), jnp.float32),
        ],
    )(class_preds[0], small)

    boxes_out = boxes_r[:4, :_MAX_DET].T
    scores_out = scores_r[0, :_MAX_DET]
    classes_out = classes_r[0, :_MAX_DET]
    return boxes_out, scores_out, classes_out
